# baseline (speedup 1.0000x reference)
import functools

import jax
import jax.numpy as jnp
from jax import lax
from jax.experimental import pallas as pl
from jax.experimental.pallas import tpu as pltpu
from jax.experimental.pallas import tpu_sc as plsc

N = 65536
K = 1024
D = 32
COMMITMENT_COST = 0.25

NCHUNK = 2
NC = N // NCHUNK
B = 2048
G = NC // B


def _vq_tc_body(x_ref, wt_ref, sse_in_ref, cnt_in_ref,
                idx_ref, o1_ref, o2_ref, cnt_ref, *, final):
    step = pl.program_id(0)

    x = x_ref[...]
    wt = wt_ref[...]

    x2 = jnp.sum(x * x, axis=1, keepdims=True)
    e2 = jnp.sum(wt * wt, axis=0, keepdims=True)
    m2 = lax.dot_general(x, wt + wt, (((1,), (0,)), ((), ())),
                         preferred_element_type=jnp.float32)
    d = (x2 + e2) - m2

    dmin = jnp.min(d, axis=1, keepdims=True)
    iotaf = lax.broadcasted_iota(jnp.int32, (1, K), 1).astype(jnp.float32)
    maskedf = jnp.where(d == dmin, iotaf, float(K))
    idxf = jnp.min(maskedf, axis=1)
    idx_ref[0, 0, :] = idxf.astype(jnp.int32)

    onehot = (iotaf == idxf[:, None]).astype(jnp.float32)
    cnt = lax.dot_general(jnp.ones((1, B), jnp.float32), onehot,
                          (((1,), (0,)), ((), ())),
                          preferred_element_type=jnp.float32)

    @pl.when(step == 0)
    def _init():
        o1_ref[...] = sse_in_ref[...]
        cnt_ref[...] = cnt_in_ref[...]

    o1_ref[...] += jnp.sum(dmin)[None, None]
    cnt_ref[...] += cnt

    @pl.when(step == G - 1)
    def _finalize():
        if final:
            o1_ref[...] = o1_ref[...] * ((1.0 + COMMITMENT_COST) / (N * D))
            avg = cnt_ref[...] * (1.0 / N)
            ent = -jnp.sum(avg * jnp.log(avg + 1e-10))
            o2_ref[...] = (jnp.exp(ent) * (1.0 / K))[None, None]
        else:
            o2_ref[...] = cnt_ref[...]


def _vq_stats_chunk(xc, wt, sse_in, cnt_in, final):
    o2_shape = (1, 1) if final else (1, K)
    return pl.pallas_call(
        functools.partial(_vq_tc_body, final=final),
        grid=(G,),
        in_specs=[
            pl.BlockSpec((B, D), lambda i: (i, 0)),
            pl.BlockSpec((D, K), lambda i: (0, 0)),
            pl.BlockSpec((1, 1), lambda i: (0, 0)),
            pl.BlockSpec((1, K), lambda i: (0, 0)),
        ],
        out_specs=[
            pl.BlockSpec((1, 1, B), lambda i: (i, 0, 0)),
            pl.BlockSpec((1, 1), lambda i: (0, 0)),
            pl.BlockSpec(o2_shape, lambda i: (0, 0)),
        ],
        out_shape=[
            jax.ShapeDtypeStruct((G, 1, B), jnp.int32),
            jax.ShapeDtypeStruct((1, 1), jnp.float32),
            jax.ShapeDtypeStruct(o2_shape, jnp.float32),
        ],
        scratch_shapes=[pltpu.VMEM((1, K), jnp.float32)],
        compiler_params=pltpu.CompilerParams(
            dimension_semantics=("arbitrary",)),
    )(xc, wt, sse_in, cnt_in)


def _make_gather():
    info = plsc.get_sparse_core_info()
    nc, ns = info.num_cores, info.num_subcores
    nw = nc * ns
    bpw = NC // nw
    mesh = plsc.VectorSubcoreMesh(core_axis_name="c", subcore_axis_name="s")

    @functools.partial(
        pl.kernel, mesh=mesh,
        out_type=jax.ShapeDtypeStruct((NC, D), jnp.float32),
        scratch_types=[
            pltpu.VMEM((bpw,), jnp.int32),
            pltpu.VMEM((bpw, D), jnp.float32),
            pltpu.SemaphoreType.DMA,
        ],
        compiler_params=pltpu.CompilerParams(use_tc_tiling_on_sc=False),
    )
    def gather_k(table_hbm, idx_hbm, out_hbm, idx_v, rows_v, sem):
        wid = lax.axis_index("s") * nc + lax.axis_index("c")
        base = wid * bpw
        pltpu.sync_copy(idx_hbm.at[pl.ds(base, bpw)], idx_v)
        pltpu.async_copy(table_hbm.at[idx_v], rows_v, sem).wait()
        pltpu.sync_copy(rows_v, out_hbm.at[pl.ds(base, bpw)])

    return gather_k


def kernel(inputs, weight):
    wt = weight.T
    gather = _make_gather()

    sse = jnp.zeros((1, 1), jnp.float32)
    cnt = jnp.zeros((1, K), jnp.float32)

    idx0, sse, cnt = _vq_stats_chunk(inputs[:NC], wt, sse, cnt, final=False)
    idx0_flat = idx0.reshape(NC)
    q0 = gather(weight, idx0_flat)

    idx1, loss, perp = _vq_stats_chunk(inputs[NC:], wt, sse, cnt, final=True)
    idx1_flat = idx1.reshape(NC)
    q1 = gather(weight, idx1_flat)

    quantized = jnp.concatenate([q0, q1], axis=0)
    idx_flat = jnp.concatenate([idx0_flat, idx1_flat])
    return (loss[0, 0], quantized, perp[0, 0], idx_flat[:, None])


# speedup vs baseline: 1.1282x; 1.1282x over previous
import functools

import jax
import jax.numpy as jnp
from jax import lax
from jax.experimental import pallas as pl
from jax.experimental.pallas import tpu as pltpu
from jax.experimental.pallas import tpu_sc as plsc

N = 65536
K = 1024
D = 32
COMMITMENT_COST = 0.25

B = 4096
G = N // B


def _vq_tc_body(x_ref, wt_ref, idx_ref, loss_ref, perp_ref, cnt_ref):
    step = pl.program_id(0)

    x = x_ref[...]
    wt = wt_ref[...]

    x2 = jnp.sum(x * x, axis=1, keepdims=True)
    e2 = jnp.sum(wt * wt, axis=0, keepdims=True)
    m2 = lax.dot_general(x, wt + wt, (((1,), (0,)), ((), ())),
                         preferred_element_type=jnp.float32)
    d = (x2 + e2) - m2

    dmin = jnp.min(d, axis=1, keepdims=True)
    iotaf = lax.broadcasted_iota(jnp.int32, (1, K), 1).astype(jnp.float32)
    maskedf = jnp.where(d == dmin, iotaf, float(K))
    idxf = jnp.min(maskedf, axis=1)
    idx_ref[0, 0, :] = idxf.astype(jnp.int32)

    onehot = (iotaf == idxf[:, None]).astype(jnp.float32)
    cnt = lax.dot_general(jnp.ones((1, B), jnp.float32), onehot,
                          (((1,), (0,)), ((), ())),
                          preferred_element_type=jnp.float32)

    @pl.when(step == 0)
    def _init():
        loss_ref[...] = jnp.zeros_like(loss_ref)
        cnt_ref[...] = jnp.zeros_like(cnt_ref)

    loss_ref[...] += jnp.sum(dmin)[None, None]
    cnt_ref[...] += cnt

    @pl.when(step == G - 1)
    def _finalize():
        loss_ref[...] = loss_ref[...] * ((1.0 + COMMITMENT_COST) / (N * D))
        avg = cnt_ref[...] * (1.0 / N)
        ent = -jnp.sum(avg * jnp.log(avg + 1e-10))
        perp_ref[...] = (jnp.exp(ent) * (1.0 / K))[None, None]


def _vq_stats(inputs, wt, interpret=False):
    return pl.pallas_call(
        _vq_tc_body,
        grid=(G,),
        in_specs=[
            pl.BlockSpec((B, D), lambda i: (i, 0)),
            pl.BlockSpec((D, K), lambda i: (0, 0)),
        ],
        out_specs=[
            pl.BlockSpec((1, 1, B), lambda i: (i, 0, 0)),
            pl.BlockSpec((1, 1), lambda i: (0, 0)),
            pl.BlockSpec((1, 1), lambda i: (0, 0)),
        ],
        out_shape=[
            jax.ShapeDtypeStruct((G, 1, B), jnp.int32),
            jax.ShapeDtypeStruct((1, 1), jnp.float32),
            jax.ShapeDtypeStruct((1, 1), jnp.float32),
        ],
        scratch_shapes=[pltpu.VMEM((1, K), jnp.float32)],
        compiler_params=pltpu.CompilerParams(
            dimension_semantics=("arbitrary",)),
        interpret=interpret,
    )(inputs, wt)


def _make_gather():
    info = plsc.get_sparse_core_info()
    nc, ns = info.num_cores, info.num_subcores
    nw = nc * ns
    bpw = N // nw
    mesh = plsc.VectorSubcoreMesh(core_axis_name="c", subcore_axis_name="s")

    @functools.partial(
        pl.kernel, mesh=mesh,
        out_type=jax.ShapeDtypeStruct((N, D), jnp.float32),
        scratch_types=[
            pltpu.VMEM((bpw,), jnp.int32),
            pltpu.VMEM((bpw, D), jnp.float32),
            pltpu.SemaphoreType.DMA,
        ],
        compiler_params=pltpu.CompilerParams(use_tc_tiling_on_sc=False),
    )
    def gather_k(table_hbm, idx_hbm, out_hbm, idx_v, rows_v, sem):
        wid = lax.axis_index("s") * nc + lax.axis_index("c")
        base = wid * bpw
        pltpu.sync_copy(idx_hbm.at[pl.ds(base, bpw)], idx_v)
        pltpu.async_copy(table_hbm.at[idx_v], rows_v, sem).wait()
        pltpu.sync_copy(rows_v, out_hbm.at[pl.ds(base, bpw)])

    return gather_k


def kernel(inputs, weight):
    wt = weight.T
    idx3, loss, perp = _vq_stats(inputs, wt)
    idx_flat = idx3.reshape(N)
    quantized = _make_gather()(weight, idx_flat)
    return (loss[0, 0], quantized, perp[0, 0], idx_flat[:, None])


# speedup vs baseline: 1.1384x; 1.0091x over previous
import functools

import jax
import jax.numpy as jnp
from jax import lax
from jax.experimental import pallas as pl
from jax.experimental.pallas import tpu as pltpu
from jax.experimental.pallas import tpu_sc as plsc

N = 65536
K = 1024
D = 32
COMMITMENT_COST = 0.25

B = 8192
G = N // B


def _vq_tc_body(x_ref, wt_ref, idx_ref, loss_ref, perp_ref, cnt_ref):
    step = pl.program_id(0)

    x = x_ref[...]
    wt = wt_ref[...]

    x2 = jnp.sum(x * x, axis=1, keepdims=True)
    e2 = jnp.sum(wt * wt, axis=0, keepdims=True)
    m2 = lax.dot_general(x, wt + wt, (((1,), (0,)), ((), ())),
                         preferred_element_type=jnp.float32)
    d = (x2 + e2) - m2

    dmin = jnp.min(d, axis=1, keepdims=True)
    iotaf = lax.broadcasted_iota(jnp.int32, (1, K), 1).astype(jnp.float32)
    maskedf = jnp.where(d == dmin, iotaf, float(K))
    idxf = jnp.min(maskedf, axis=1)
    idx_ref[0, 0, :] = idxf.astype(jnp.int32)

    onehot = (iotaf == idxf[:, None]).astype(jnp.float32)
    cnt = lax.dot_general(jnp.ones((1, B), jnp.float32), onehot,
                          (((1,), (0,)), ((), ())),
                          preferred_element_type=jnp.float32)

    @pl.when(step == 0)
    def _init():
        loss_ref[...] = jnp.zeros_like(loss_ref)
        cnt_ref[...] = jnp.zeros_like(cnt_ref)

    loss_ref[...] += jnp.sum(dmin)[None, None]
    cnt_ref[...] += cnt

    @pl.when(step == G - 1)
    def _finalize():
        loss_ref[...] = loss_ref[...] * ((1.0 + COMMITMENT_COST) / (N * D))
        avg = cnt_ref[...] * (1.0 / N)
        ent = -jnp.sum(avg * jnp.log(avg + 1e-10))
        perp_ref[...] = (jnp.exp(ent) * (1.0 / K))[None, None]


def _vq_stats(inputs, wt, interpret=False):
    return pl.pallas_call(
        _vq_tc_body,
        grid=(G,),
        in_specs=[
            pl.BlockSpec((B, D), lambda i: (i, 0)),
            pl.BlockSpec((D, K), lambda i: (0, 0)),
        ],
        out_specs=[
            pl.BlockSpec((1, 1, B), lambda i: (i, 0, 0)),
            pl.BlockSpec((1, 1), lambda i: (0, 0)),
            pl.BlockSpec((1, 1), lambda i: (0, 0)),
        ],
        out_shape=[
            jax.ShapeDtypeStruct((G, 1, B), jnp.int32),
            jax.ShapeDtypeStruct((1, 1), jnp.float32),
            jax.ShapeDtypeStruct((1, 1), jnp.float32),
        ],
        scratch_shapes=[pltpu.VMEM((1, K), jnp.float32)],
        compiler_params=pltpu.CompilerParams(
            dimension_semantics=("arbitrary",)),
        interpret=interpret,
    )(inputs, wt)


def _make_gather():
    info = plsc.get_sparse_core_info()
    nc, ns = info.num_cores, info.num_subcores
    nw = nc * ns
    bpw = N // nw
    mesh = plsc.VectorSubcoreMesh(core_axis_name="c", subcore_axis_name="s")

    @functools.partial(
        pl.kernel, mesh=mesh,
        out_type=jax.ShapeDtypeStruct((N, D), jnp.float32),
        scratch_types=[
            pltpu.VMEM((bpw,), jnp.int32),
            pltpu.VMEM((bpw, D), jnp.float32),
            pltpu.SemaphoreType.DMA,
        ],
        compiler_params=pltpu.CompilerParams(use_tc_tiling_on_sc=False),
    )
    def gather_k(table_hbm, idx_hbm, out_hbm, idx_v, rows_v, sem):
        wid = lax.axis_index("s") * nc + lax.axis_index("c")
        base = wid * bpw
        pltpu.sync_copy(idx_hbm.at[pl.ds(base, bpw)], idx_v)
        pltpu.async_copy(table_hbm.at[idx_v], rows_v, sem).wait()
        pltpu.sync_copy(rows_v, out_hbm.at[pl.ds(base, bpw)])

    return gather_k


def kernel(inputs, weight):
    wt = weight.T
    idx3, loss, perp = _vq_stats(inputs, wt)
    idx_flat = idx3.reshape(N)
    quantized = _make_gather()(weight, idx_flat)
    return (loss[0, 0], quantized, perp[0, 0], idx_flat[:, None])
